# baseline (speedup 1.0000x reference)
import functools

import jax
import jax.numpy as jnp
from jax import lax
from jax.experimental import pallas as pl
from jax.experimental.pallas import tpu as pltpu
from jax.experimental.pallas import tpu_sc as plsc

NN = 100000
NSEG = 1000
OB = 1024
NC, NS = 2, 16
NW = NC * NS
CH = 3200
NV = CH // 16
SPW = 128
INF = float("inf")
IMAX = 2147483647

_mesh = plsc.VectorSubcoreMesh(
    core_axis_name="c", subcore_axis_name="s", num_cores=NC, num_subcores=NS)


def _iota16():
    return lax.broadcasted_iota(jnp.int32, (16,), 0)


def _bc(x):
    return jnp.broadcast_to(x, (16,))


@functools.partial(
    pl.kernel,
    out_type=(
        jax.ShapeDtypeStruct((NW, OB), jnp.float32),
        jax.ShapeDtypeStruct((NW, OB), jnp.int32),
    ),
    mesh=_mesh,
    compiler_params=pltpu.CompilerParams(
        use_tc_tiling_on_sc=False, needs_layout_passes=False),
    scratch_types=[
        pltpu.VMEM((CH, 8), jnp.float32),
        pltpu.VMEM((CH,), jnp.float32),
        pltpu.VMEM((CH,), jnp.int32),
        pltpu.VMEM((OB,), jnp.float32),
        pltpu.VMEM((OB,), jnp.int32),
        pltpu.SemaphoreType.DMA,
        pltpu.SemaphoreType.DMA,
    ],
)
def _phase1(h_hbm, chi_hbm, bidx_hbm, pval_hbm, pidx_hbm,
            h16_v, chi_v, bidx_v, oval_v, oidx_v, sema, semb):
    wid = lax.axis_index("s") * NC + lax.axis_index("c")
    base = jnp.minimum(wid * CH, NN - CH)
    lanes = _iota16()
    lane0 = lanes == 0

    cpa = pltpu.async_copy(
        h_hbm.at[pl.ds(base, CH), pl.ds(0, 8)], h16_v, sema)
    cpc = pltpu.async_copy(chi_hbm.at[pl.ds(base, CH)], chi_v, semb)
    cpb = pltpu.async_copy(bidx_hbm.at[pl.ds(base, CH)], bidx_v, semb)

    inf_vec = jnp.full((16,), INF, jnp.float32)
    big_idx = jnp.full((16,), NN, jnp.int32)

    def init(i, _):
        oval_v[pl.ds(i * 16, 16)] = inf_vec
        oidx_v[pl.ds(i * 16, 16)] = big_idx
        return 0

    lax.fori_loop(0, OB // 16, init, 0)

    def flush(cs, av, an):
        m = jnp.min(av)
        mvec = _bc(m)
        nodemin = jnp.min(jnp.where(av == mvec, an, NN))
        csv = _bc(cs)
        wm = lane0 & (mvec < INF)
        plsc.store_scatter(oval_v, [csv], mvec, mask=wm)
        plsc.store_scatter(oidx_v, [csv], _bc(nodemin), mask=wm)

    def step(j, carry):
        cs, av, an = carry
        off = j * 16
        vb = bidx_v[pl.ds(off, 16)]
        vc = chi_v[pl.ds(off, 16)]
        rows = _bc(off) + lanes
        h3 = plsc.load_gather(h16_v, [rows, _bc(jnp.int32(3))])
        h4 = plsc.load_gather(h16_v, [rows, _bc(jnp.int32(4))])
        h5 = plsc.load_gather(h16_v, [rows, _bc(jnp.int32(5))])
        h6 = plsc.load_gather(h16_v, [rows, _bc(jnp.int32(6))])
        filt = (h4 > h3) & (h4 >= h5) & (h4 >= h6)
        key = jnp.where(filt, vc, INF)
        node = _bc(base + off) + lanes
        vb15 = vb[15]

        mc0 = vb == _bc(cs)
        upd0 = mc0 & (key < av)
        av = jnp.where(upd0, key, av)
        an = jnp.where(upd0, node, an)

        def fast(carry):
            return carry

        def slow(carry):
            cs, av, an = carry
            flush(cs, av, an)

            def cond(carry):
                rem, cs, av, an = carry
                return jnp.any(rem)

            def body(carry):
                rem, cs, av, an = carry
                s = jnp.min(jnp.where(rem, vb, IMAX))
                svec = _bc(s)
                segm = vb == svec
                kseg = jnp.where(segm, key, INF)
                is_last = s == vb15

                def mid(args):
                    cs, av, an = args
                    m = jnp.min(kseg)
                    mvec = _bc(m)
                    nodemin = jnp.min(
                        jnp.where(segm & (kseg == mvec), node, NN))
                    wm = lane0 & (mvec < INF)
                    plsc.store_scatter(oval_v, [svec], mvec, mask=wm)
                    plsc.store_scatter(oidx_v, [svec], _bc(nodemin), mask=wm)
                    return cs, av, an

                def last(args):
                    return s, kseg, node

                cs, av, an = lax.cond(is_last, last, mid, (cs, av, an))
                return rem & ~segm, cs, av, an

            rem0 = ~(vb == _bc(cs))
            _, cs, av, an = lax.while_loop(cond, body, (rem0, cs, av, an))
            return cs, av, an

        return lax.cond(vb15 == cs, fast, slow, (cs, av, an))

    cpc.wait()
    cpb.wait()
    cs0 = bidx_v[pl.ds(0, 16)][0]
    carry = (cs0, jnp.full((16,), INF, jnp.float32),
             jnp.zeros((16,), jnp.int32))
    cpa.wait()

    def step2(jj, carry):
        return step(jj * 2 + 1, step(jj * 2, carry))

    carry = lax.fori_loop(0, NV // 2, step2, carry)
    flush(*carry)

    pltpu.sync_copy(oval_v, pval_hbm.at[wid])
    pltpu.sync_copy(oidx_v, pidx_hbm.at[wid])


@functools.partial(
    pl.kernel,
    out_type=(
        jax.ShapeDtypeStruct((OB,), jnp.float32),
        jax.ShapeDtypeStruct((OB * 3,), jnp.float32),
    ),
    mesh=_mesh,
    compiler_params=pltpu.CompilerParams(
        use_tc_tiling_on_sc=False, needs_layout_passes=False),
    scratch_types=[
        pltpu.VMEM((NW, SPW), jnp.float32),
        pltpu.VMEM((NW, SPW), jnp.int32),
        pltpu.VMEM((3, SPW), jnp.int32),
        pltpu.VMEM((3, SPW), jnp.float32),
        pltpu.VMEM((SPW * 3,), jnp.float32),
        pltpu.VMEM((SPW,), jnp.float32),
        pltpu.SemaphoreType.DMA,
    ],
)
def _phase2(pval_hbm, pidx_hbm, posf_hbm, ptr_hbm, pdir_hbm,
            pv_v, pi_v, gidx_v, pbuf_v, pdir_v, ptr_v, semg):
    wid = lax.axis_index("s") * NC + lax.axis_index("c")

    @pl.when(wid < 8)
    def _():
        t = wid
        seg0 = t * SPW
        lanes = _iota16()

        pltpu.sync_copy(pval_hbm.at[:, pl.ds(seg0, SPW)], pv_v)
        pltpu.sync_copy(pidx_hbm.at[:, pl.ds(seg0, SPW)], pi_v)

        def combine(k, _):
            sl = pl.ds(k * 16, 16)
            bv = jnp.full((16,), INF, jnp.float32)
            bi = jnp.full((16,), NN, jnp.int32)
            for w in range(NW):
                av = pv_v[w, sl]
                ai = pi_v[w, sl]
                better = (av < bv) | ((av == bv) & (ai < bi))
                bv = jnp.where(better, av, bv)
                bi = jnp.where(better, ai, bi)
            pickf = jnp.where(bv < INF, bi, 0)
            gidx_v[0, sl] = pickf
            gidx_v[1, sl] = pickf + NN
            gidx_v[2, sl] = pickf + 2 * NN
            return 0

        lax.fori_loop(0, SPW // 16, combine, 0)

        cps = [
            pltpu.async_copy(
                posf_hbm.at[gidx_v.at[c]], pbuf_v.at[c], semg)
            for c in range(3)
        ]
        for cp in cps:
            cp.wait()

        def norm_step(k, _):
            sl = pl.ds(k * 16, 16)
            x = pbuf_v[0, sl]
            y = pbuf_v[1, sl]
            z = pbuf_v[2, sl]
            s = x * x + y * y + z * z
            i = plsc.bitcast(s, jnp.int32)
            i = jnp.int32(0x1FBD1DF5) + (i >> 1)
            r = plsc.bitcast(i, jnp.float32)
            r = 0.5 * (r + s / r)
            r = 0.5 * (r + s / r)
            r = 0.5 * (r + s / r)
            r = jnp.where(s > 0.0, r, 0.0)
            ptr_v[sl] = r
            kv = (_bc(k * 16) + lanes) * 3
            plsc.store_scatter(pdir_v, [kv], x)
            plsc.store_scatter(pdir_v, [kv + 1], y)
            plsc.store_scatter(pdir_v, [kv + 2], z)
            return 0

        lax.fori_loop(0, SPW // 16, norm_step, 0)

        pltpu.sync_copy(ptr_v, ptr_hbm.at[pl.ds(seg0, SPW)])
        pltpu.sync_copy(pdir_v, pdir_hbm.at[pl.ds(seg0 * 3, SPW * 3)])


def kernel(x_global_features, h, pos_pxpypz_at_vertex, chi_squared_tracks, batch_idx):
    del x_global_features
    posf = jnp.ravel(pos_pxpypz_at_vertex.T)
    pval, pidx = _phase1(h, chi_squared_tracks, batch_idx.astype(jnp.int32))
    p_tracks, pdir_flat = _phase2(pval, pidx, posf)
    return p_tracks[:NSEG], jnp.reshape(pdir_flat[:NSEG * 3], (NSEG, 3))


# speedup vs baseline: 1.0785x; 1.0785x over previous
import functools

import jax
import jax.numpy as jnp
from jax import lax
from jax.experimental import pallas as pl
from jax.experimental.pallas import tpu as pltpu
from jax.experimental.pallas import tpu_sc as plsc

NN = 100000
NSEG = 1000
OB = 1024
NC, NS = 2, 16
NW = NC * NS
CH = 3200
NV = CH // 16
SPW = 128
INF = float("inf")
IMAX = 2147483647

_mesh = plsc.VectorSubcoreMesh(
    core_axis_name="c", subcore_axis_name="s", num_cores=NC, num_subcores=NS)


def _iota16():
    return lax.broadcasted_iota(jnp.int32, (16,), 0)


def _bc(x):
    return jnp.broadcast_to(x, (16,))


@functools.partial(
    pl.kernel,
    out_type=(
        jax.ShapeDtypeStruct((NW, OB), jnp.float32),
        jax.ShapeDtypeStruct((NW, OB), jnp.int32),
    ),
    mesh=_mesh,
    compiler_params=pltpu.CompilerParams(
        use_tc_tiling_on_sc=False, needs_layout_passes=False),
    scratch_types=[
        pltpu.VMEM((CH, 8), jnp.float32),
        pltpu.VMEM((CH,), jnp.float32),
        pltpu.VMEM((CH,), jnp.int32),
        pltpu.VMEM((OB,), jnp.float32),
        pltpu.VMEM((OB,), jnp.int32),
        pltpu.SemaphoreType.DMA,
        pltpu.SemaphoreType.DMA,
    ],
)
def _phase1(h_hbm, chi_hbm, bidx_hbm, pval_hbm, pidx_hbm,
            h16_v, chi_v, bidx_v, oval_v, oidx_v, sema, semb):
    wid = lax.axis_index("s") * NC + lax.axis_index("c")
    base = jnp.minimum(wid * CH, NN - CH)
    lanes = _iota16()
    lane0 = lanes == 0

    cpa = pltpu.async_copy(
        h_hbm.at[pl.ds(base, CH), pl.ds(0, 8)], h16_v, sema)
    cpc = pltpu.async_copy(chi_hbm.at[pl.ds(base, CH)], chi_v, semb)
    cpb = pltpu.async_copy(bidx_hbm.at[pl.ds(base, CH)], bidx_v, semb)

    inf_vec = jnp.full((16,), INF, jnp.float32)
    big_idx = jnp.full((16,), NN, jnp.int32)

    def init(i, _):
        oval_v[pl.ds(i * 16, 16)] = inf_vec
        oidx_v[pl.ds(i * 16, 16)] = big_idx
        return 0

    lax.fori_loop(0, OB // 16, init, 0)

    def flush(cs, av, an):
        m = jnp.min(av)
        mvec = _bc(m)
        nodemin = jnp.min(jnp.where(av == mvec, an, NN))
        csv = _bc(cs)
        wm = lane0 & (mvec < INF)
        plsc.store_scatter(oval_v, [csv], mvec, mask=wm)
        plsc.store_scatter(oidx_v, [csv], _bc(nodemin), mask=wm)

    def step(j, carry):
        cs, av, an = carry
        off = j * 16
        vb = bidx_v[pl.ds(off, 16)]
        vc = chi_v[pl.ds(off, 16)]
        rows = _bc(off) + lanes
        h3 = plsc.load_gather(h16_v, [rows, _bc(jnp.int32(3))])
        h4 = plsc.load_gather(h16_v, [rows, _bc(jnp.int32(4))])
        h5 = plsc.load_gather(h16_v, [rows, _bc(jnp.int32(5))])
        h6 = plsc.load_gather(h16_v, [rows, _bc(jnp.int32(6))])
        filt = (h4 > h3) & (h4 >= h5) & (h4 >= h6)
        key = jnp.where(filt, vc, INF)
        node = _bc(base + off) + lanes
        vb0 = vb[0]
        vb15 = vb[15]

        def fast(carry):
            cs, av, an = carry
            upd = key < av
            return cs, jnp.minimum(av, key), jnp.where(upd, node, an)

        def slow(carry):
            cs, av, an = carry
            csv = _bc(cs)
            mc = vb == csv
            upd = mc & (key < av)
            av = jnp.where(upd, key, av)
            an = jnp.where(upd, node, an)
            flush(cs, av, an)

            def cond(carry):
                rem, cs, av, an = carry
                return jnp.any(rem)

            def body(carry):
                rem, cs, av, an = carry
                s = jnp.min(jnp.where(rem, vb, IMAX))
                svec = _bc(s)
                segm = vb == svec
                kseg = jnp.where(segm, key, INF)
                is_last = s == vb15

                def mid(args):
                    cs, av, an = args
                    m = jnp.min(kseg)
                    mvec = _bc(m)
                    nodemin = jnp.min(
                        jnp.where(segm & (kseg == mvec), node, NN))
                    wm = lane0 & (mvec < INF)
                    plsc.store_scatter(oval_v, [svec], mvec, mask=wm)
                    plsc.store_scatter(oidx_v, [svec], _bc(nodemin), mask=wm)
                    return cs, av, an

                def last(args):
                    return s, kseg, node

                cs, av, an = lax.cond(is_last, last, mid, (cs, av, an))
                return rem & ~segm, cs, av, an

            rem0 = ~mc
            _, cs, av, an = lax.while_loop(cond, body, (rem0, cs, av, an))
            return cs, av, an

        is_fast = (vb0 == vb15) & (vb0 == cs)
        return lax.cond(is_fast, fast, slow, (cs, av, an))

    cpc.wait()
    cpb.wait()
    cs0 = bidx_v[pl.ds(0, 16)][0]
    carry = (cs0, jnp.full((16,), INF, jnp.float32),
             jnp.zeros((16,), jnp.int32))
    cpa.wait()

    def step2(jj, carry):
        return step(jj * 2 + 1, step(jj * 2, carry))

    carry = lax.fori_loop(0, NV // 2, step2, carry)
    flush(*carry)

    pltpu.sync_copy(oval_v, pval_hbm.at[wid])
    pltpu.sync_copy(oidx_v, pidx_hbm.at[wid])


@functools.partial(
    pl.kernel,
    out_type=(
        jax.ShapeDtypeStruct((OB,), jnp.float32),
        jax.ShapeDtypeStruct((OB * 3,), jnp.float32),
    ),
    mesh=_mesh,
    compiler_params=pltpu.CompilerParams(
        use_tc_tiling_on_sc=False, needs_layout_passes=False),
    scratch_types=[
        pltpu.VMEM((NW, SPW), jnp.float32),
        pltpu.VMEM((NW, SPW), jnp.int32),
        pltpu.VMEM((3, SPW), jnp.int32),
        pltpu.VMEM((3, SPW), jnp.float32),
        pltpu.VMEM((SPW * 3,), jnp.float32),
        pltpu.VMEM((SPW,), jnp.float32),
        pltpu.SemaphoreType.DMA,
    ],
)
def _phase2(pval_hbm, pidx_hbm, posf_hbm, ptr_hbm, pdir_hbm,
            pv_v, pi_v, gidx_v, pbuf_v, pdir_v, ptr_v, semg):
    wid = lax.axis_index("s") * NC + lax.axis_index("c")

    @pl.when(wid < 8)
    def _():
        t = wid
        seg0 = t * SPW
        lanes = _iota16()

        pltpu.sync_copy(pval_hbm.at[:, pl.ds(seg0, SPW)], pv_v)
        pltpu.sync_copy(pidx_hbm.at[:, pl.ds(seg0, SPW)], pi_v)

        def combine(k, _):
            sl = pl.ds(k * 16, 16)
            bv = jnp.full((16,), INF, jnp.float32)
            bi = jnp.full((16,), NN, jnp.int32)
            for w in range(NW):
                av = pv_v[w, sl]
                ai = pi_v[w, sl]
                better = (av < bv) | ((av == bv) & (ai < bi))
                bv = jnp.where(better, av, bv)
                bi = jnp.where(better, ai, bi)
            pickf = jnp.where(bv < INF, bi, 0)
            gidx_v[0, sl] = pickf
            gidx_v[1, sl] = pickf + NN
            gidx_v[2, sl] = pickf + 2 * NN
            return 0

        lax.fori_loop(0, SPW // 16, combine, 0)

        cps = [
            pltpu.async_copy(
                posf_hbm.at[gidx_v.at[c]], pbuf_v.at[c], semg)
            for c in range(3)
        ]
        for cp in cps:
            cp.wait()

        def norm_step(k, _):
            sl = pl.ds(k * 16, 16)
            x = pbuf_v[0, sl]
            y = pbuf_v[1, sl]
            z = pbuf_v[2, sl]
            s = x * x + y * y + z * z
            i = plsc.bitcast(s, jnp.int32)
            i = jnp.int32(0x1FBD1DF5) + (i >> 1)
            r = plsc.bitcast(i, jnp.float32)
            r = 0.5 * (r + s / r)
            r = 0.5 * (r + s / r)
            r = 0.5 * (r + s / r)
            r = jnp.where(s > 0.0, r, 0.0)
            ptr_v[sl] = r
            kv = (_bc(k * 16) + lanes) * 3
            plsc.store_scatter(pdir_v, [kv], x)
            plsc.store_scatter(pdir_v, [kv + 1], y)
            plsc.store_scatter(pdir_v, [kv + 2], z)
            return 0

        lax.fori_loop(0, SPW // 16, norm_step, 0)

        pltpu.sync_copy(ptr_v, ptr_hbm.at[pl.ds(seg0, SPW)])
        pltpu.sync_copy(pdir_v, pdir_hbm.at[pl.ds(seg0 * 3, SPW * 3)])


def kernel(x_global_features, h, pos_pxpypz_at_vertex, chi_squared_tracks, batch_idx):
    del x_global_features
    posf = jnp.ravel(pos_pxpypz_at_vertex.T)
    pval, pidx = _phase1(h, chi_squared_tracks, batch_idx.astype(jnp.int32))
    p_tracks, pdir_flat = _phase2(pval, pidx, posf)
    return p_tracks[:NSEG], jnp.reshape(pdir_flat[:NSEG * 3], (NSEG, 3))
